# baseline (speedup 1.0000x reference)
import jax
import jax.numpy as jnp
from jax.experimental import pallas as pl
from jax.experimental.pallas import tpu as pltpu

_N, _D, _H, _GH, _E = 2048, 768, 128, 64, 16
_BN = 512
_NB = _N // _BN
_BALANCE_COEF = 0.01


def _gate_body(x_ref, gw1_ref, gb1_ref, gw2_ref, gb2_ref,
               combine_ref, usage_ref, loss_ref):
    x = x_ref[...]
    gh = jnp.maximum(
        jnp.dot(x, gw1_ref[...], preferred_element_type=jnp.float32)
        + gb1_ref[...], 0.0)
    logits = (jnp.dot(gh, gw2_ref[...], preferred_element_type=jnp.float32)
              + gb2_ref[...])
    eid = jax.lax.broadcasted_iota(jnp.int32, logits.shape, 1)
    l1 = jnp.max(logits, axis=1, keepdims=True)
    i1 = jnp.min(jnp.where(logits == l1, eid, _E), axis=1, keepdims=True)
    m1 = eid == i1
    masked = jnp.where(m1, jnp.float32(-1e30), logits)
    l2 = jnp.max(masked, axis=1, keepdims=True)
    i2 = jnp.min(jnp.where(masked == l2, eid, _E), axis=1, keepdims=True)
    m2 = eid == i2
    wa = 1.0 / (1.0 + jnp.exp(l2 - l1))
    combine_ref[...] = jnp.where(m1, wa, 0.0) + jnp.where(m2, 1.0 - wa, 0.0)
    usage = jnp.sum((m1 | m2).astype(jnp.float32), axis=0,
                    keepdims=True) * (1.0 / _N)
    usage_ref[...] = usage
    loss_ref[...] = (jnp.mean((usage - 1.0 / _E) ** 2)
                     * _BALANCE_COEF).reshape(1, 1)


def _expert_body(x_ref, c_ref, w1_ref, b1_ref, w2_ref, b2_ref,
                 w3_ref, b3_ref, out_ref, w1c, w2c, w3c):
    i = pl.program_id(0)

    @pl.when(i == 0)
    def _():
        for e in range(_E):
            w1c[:, e * _H:(e + 1) * _H] = w1_ref[e].astype(jnp.bfloat16)
            w2c[e] = w2_ref[e].astype(jnp.bfloat16)
            w3c[e * _H:(e + 1) * _H, :] = w3_ref[e].astype(jnp.bfloat16)

    xb = x_ref[...].astype(jnp.bfloat16)
    combine = c_ref[...]
    h1 = jnp.maximum(
        jnp.dot(xb, w1c[...], preferred_element_type=jnp.float32)
        + b1_ref[...], 0.0)
    h2s = [None] * _E
    for e in range(_E):
        h2e = jnp.maximum(
            jnp.dot(h1[:, e * _H:(e + 1) * _H].astype(jnp.bfloat16),
                    w2c[e], preferred_element_type=jnp.float32)
            + b2_ref[:, e * _H:(e + 1) * _H], 0.0)
        h2s[e] = h2e * combine[:, e:e + 1]
    h2s = jnp.concatenate(h2s, axis=1)
    y = jnp.dot(h2s.astype(jnp.bfloat16), w3c[...],
                preferred_element_type=jnp.float32)
    y += jnp.dot(combine, b3_ref[...], preferred_element_type=jnp.float32)
    out_ref[...] = y


def kernel(x, gate_W1, gate_b1, gate_W2, gate_b2, W1, b1, W2, b2, W3, b3):
    combine, usage, loss = pl.pallas_call(
        _gate_body,
        out_shape=(
            jax.ShapeDtypeStruct((_N, _E), jnp.float32),
            jax.ShapeDtypeStruct((1, _E), jnp.float32),
            jax.ShapeDtypeStruct((1, 1), jnp.float32),
        ),
    )(x, gate_W1, gate_b1.reshape(1, _GH), gate_W2, gate_b2.reshape(1, _E))

    out = pl.pallas_call(
        _expert_body,
        grid=(_NB,),
        in_specs=[
            pl.BlockSpec((_BN, _D), lambda i: (i, 0)),
            pl.BlockSpec((_BN, _E), lambda i: (i, 0)),
            pl.BlockSpec((_E, _D, _H), lambda i: (0, 0, 0)),
            pl.BlockSpec((1, _E * _H), lambda i: (0, 0)),
            pl.BlockSpec((_E, _H, _H), lambda i: (0, 0, 0)),
            pl.BlockSpec((1, _E * _H), lambda i: (0, 0)),
            pl.BlockSpec((_E, _H, _D), lambda i: (0, 0, 0)),
            pl.BlockSpec((_E, _D), lambda i: (0, 0)),
        ],
        out_specs=pl.BlockSpec((_BN, _D), lambda i: (i, 0)),
        out_shape=jax.ShapeDtypeStruct((_N, _D), jnp.float32),
        scratch_shapes=[
            pltpu.VMEM((_D, _E * _H), jnp.bfloat16),
            pltpu.VMEM((_E, _H, _H), jnp.bfloat16),
            pltpu.VMEM((_E * _H, _D), jnp.bfloat16),
        ],
    )(x, combine, W1, b1.reshape(1, _E * _H), W2,
      b2.reshape(1, _E * _H), W3, b3)

    return out, loss[0, 0], usage.reshape(_E)


# speedup vs baseline: 1.0528x; 1.0528x over previous
import jax
import jax.numpy as jnp
from jax.experimental import pallas as pl
from jax.experimental.pallas import tpu as pltpu

_N, _D, _H, _GH, _E = 2048, 768, 128, 64, 16
_BN = 512
_NB = _N // _BN
_BALANCE_COEF = 0.01


def _body(xf_ref, x_ref, gw1_ref, gb1_ref, gw2_ref, gb2_ref,
          w1_ref, b1_ref, w2_ref, b2_ref, w3_ref, b3_ref,
          out_ref, usage_ref, loss_ref,
          comb_s, w1c, w2c, w3c):
    i = pl.program_id(0)

    @pl.when(i == 0)
    def _():
        for e in range(_E):
            w1c[:, e * _H:(e + 1) * _H] = w1_ref[e].astype(jnp.bfloat16)
            w2c[e] = w2_ref[e].astype(jnp.bfloat16)
            w3c[e * _H:(e + 1) * _H, :] = w3_ref[e].astype(jnp.bfloat16)
        xf = xf_ref[...]
        gh = jnp.maximum(
            jnp.dot(xf, gw1_ref[...], preferred_element_type=jnp.float32)
            + gb1_ref[...], 0.0)
        logits = (jnp.dot(gh, gw2_ref[...],
                          preferred_element_type=jnp.float32)
                  + gb2_ref[...])
        eid = jax.lax.broadcasted_iota(jnp.int32, logits.shape, 1)
        l1 = jnp.max(logits, axis=1, keepdims=True)
        i1 = jnp.min(jnp.where(logits == l1, eid, _E), axis=1, keepdims=True)
        m1 = eid == i1
        masked = jnp.where(m1, jnp.float32(-1e30), logits)
        l2 = jnp.max(masked, axis=1, keepdims=True)
        i2 = jnp.min(jnp.where(masked == l2, eid, _E), axis=1, keepdims=True)
        m2 = eid == i2
        wa = 1.0 / (1.0 + jnp.exp(l2 - l1))
        comb_s[...] = jnp.where(m1, wa, 0.0) + jnp.where(m2, 1.0 - wa, 0.0)
        usage = jnp.sum((m1 | m2).astype(jnp.float32), axis=0,
                        keepdims=True) * (1.0 / _N)
        usage_ref[...] = usage
        loss_ref[...] = (jnp.mean((usage - 1.0 / _E) ** 2)
                         * _BALANCE_COEF).reshape(1, 1)

    xb = x_ref[...].astype(jnp.bfloat16)
    combine = comb_s[pl.ds(i * _BN, _BN), :]
    h1 = jnp.maximum(
        jnp.dot(xb, w1c[...], preferred_element_type=jnp.float32)
        + b1_ref[...], 0.0)
    h2s = [None] * _E
    for e in range(_E):
        h2e = jnp.maximum(
            jnp.dot(h1[:, e * _H:(e + 1) * _H].astype(jnp.bfloat16),
                    w2c[e], preferred_element_type=jnp.float32)
            + b2_ref[:, e * _H:(e + 1) * _H], 0.0)
        h2s[e] = h2e * combine[:, e:e + 1]
    h2s = jnp.concatenate(h2s, axis=1)
    y = jnp.dot(h2s.astype(jnp.bfloat16), w3c[...],
                preferred_element_type=jnp.float32)
    y += jnp.dot(combine, b3_ref[...], preferred_element_type=jnp.float32)
    out_ref[...] = y


def kernel(x, gate_W1, gate_b1, gate_W2, gate_b2, W1, b1, W2, b2, W3, b3):
    out, usage, loss = pl.pallas_call(
        _body,
        grid=(_NB,),
        in_specs=[
            pl.BlockSpec((_N, _D), lambda i: (0, 0)),
            pl.BlockSpec((_BN, _D), lambda i: (i, 0)),
            pl.BlockSpec((_D, _GH), lambda i: (0, 0)),
            pl.BlockSpec((1, _GH), lambda i: (0, 0)),
            pl.BlockSpec((_GH, _E), lambda i: (0, 0)),
            pl.BlockSpec((1, _E), lambda i: (0, 0)),
            pl.BlockSpec((_E, _D, _H), lambda i: (0, 0, 0)),
            pl.BlockSpec((1, _E * _H), lambda i: (0, 0)),
            pl.BlockSpec((_E, _H, _H), lambda i: (0, 0, 0)),
            pl.BlockSpec((1, _E * _H), lambda i: (0, 0)),
            pl.BlockSpec((_E, _H, _D), lambda i: (0, 0, 0)),
            pl.BlockSpec((_E, _D), lambda i: (0, 0)),
        ],
        out_specs=(
            pl.BlockSpec((_BN, _D), lambda i: (i, 0)),
            pl.BlockSpec((1, _E), lambda i: (0, 0)),
            pl.BlockSpec((1, 1), lambda i: (0, 0)),
        ),
        out_shape=(
            jax.ShapeDtypeStruct((_N, _D), jnp.float32),
            jax.ShapeDtypeStruct((1, _E), jnp.float32),
            jax.ShapeDtypeStruct((1, 1), jnp.float32),
        ),
        scratch_shapes=[
            pltpu.VMEM((_N, _E), jnp.float32),
            pltpu.VMEM((_D, _E * _H), jnp.bfloat16),
            pltpu.VMEM((_E, _H, _H), jnp.bfloat16),
            pltpu.VMEM((_E * _H, _D), jnp.bfloat16),
        ],
    )(x, x, gate_W1, gate_b1.reshape(1, _GH), gate_W2,
      gate_b2.reshape(1, _E), W1, b1.reshape(1, _E * _H), W2,
      b2.reshape(1, _E * _H), W3, b3)

    return out, loss[0, 0], usage.reshape(_E)
